# baseline (speedup 1.0000x reference)
import functools

import numpy as np
import jax
import jax.numpy as jnp
from jax.experimental import pallas as pl
from jax.experimental.pallas import tpu as pltpu

B = 8
T = 2048
L = 256
VOCAB = 512
P = 100
PPAD = 128
JPAD = 264
NBEST_SCALE = 0.5
TCHUNK = 256
BIG = 1 << 20


def _np_threefry2x32(k1, k2, x0, x1):
    ks0 = np.uint32(k1)
    ks1 = np.uint32(k2)
    ks2 = np.uint32(ks0 ^ ks1 ^ np.uint32(0x1BD11BDA))

    def rotl(x, d):
        return (x << np.uint32(d)) | (x >> np.uint32(32 - d))

    def rounds(x0, x1, rots):
        for r in rots:
            x0 = (x0 + x1).astype(np.uint32)
            x1 = x0 ^ rotl(x1, r)
        return x0, x1

    x0 = (x0 + ks0).astype(np.uint32)
    x1 = (x1 + ks1).astype(np.uint32)
    x0, x1 = rounds(x0, x1, (13, 15, 26, 6))
    x0 = (x0 + ks1).astype(np.uint32)
    x1 = (x1 + ks2 + np.uint32(1)).astype(np.uint32)
    x0, x1 = rounds(x0, x1, (17, 29, 16, 24))
    x0 = (x0 + ks2).astype(np.uint32)
    x1 = (x1 + ks0 + np.uint32(2)).astype(np.uint32)
    x0, x1 = rounds(x0, x1, (13, 15, 26, 6))
    x0 = (x0 + ks0).astype(np.uint32)
    x1 = (x1 + ks1 + np.uint32(3)).astype(np.uint32)
    x0, x1 = rounds(x0, x1, (17, 29, 16, 24))
    x0 = (x0 + ks1).astype(np.uint32)
    x1 = (x1 + ks2 + np.uint32(4)).astype(np.uint32)
    x0, x1 = rounds(x0, x1, (13, 15, 26, 6))
    x0 = (x0 + ks2).astype(np.uint32)
    x1 = (x1 + ks0 + np.uint32(5)).astype(np.uint32)
    return x0, x1


def _path_subkeys():
    ks = []
    k = (np.uint32(0), np.uint32(42))
    for _ in range(P):
        b1, b2 = _np_threefry2x32(k[0], k[1],
                                  np.zeros(2, np.uint32),
                                  np.arange(2, dtype=np.uint32))
        k = (b1[0], b2[0])
        ks.append((b1[1], b2[1]))
    return np.asarray(ks, dtype=np.uint32)


_SUBKEYS = _path_subkeys()


def _threefry_bits(k1, k2, x1):
    ks0 = k1
    ks1 = k2
    ks2 = k1 ^ k2 ^ jnp.uint32(0x1BD11BDA)

    def rotl(x, d):
        return (x << jnp.uint32(d)) | (x >> jnp.uint32(32 - d))

    def rounds(x0, x1, rots):
        for r in rots:
            x0 = x0 + x1
            x1 = x0 ^ rotl(x1, r)
        return x0, x1

    x0 = jnp.broadcast_to(ks0, x1.shape)
    x1 = x1 + ks1
    x0, x1 = rounds(x0, x1, (13, 15, 26, 6))
    x0 = x0 + ks1
    x1 = x1 + (ks2 + jnp.uint32(1))
    x0, x1 = rounds(x0, x1, (17, 29, 16, 24))
    x0 = x0 + ks2
    x1 = x1 + (ks0 + jnp.uint32(2))
    x0, x1 = rounds(x0, x1, (13, 15, 26, 6))
    x0 = x0 + ks0
    x1 = x1 + (ks1 + jnp.uint32(3))
    x0, x1 = rounds(x0, x1, (17, 29, 16, 24))
    x0 = x0 + ks1
    x1 = x1 + (ks2 + jnp.uint32(4))
    x0, x1 = rounds(x0, x1, (13, 15, 26, 6))
    x0 = x0 + ks2
    x1 = x1 + (ks0 + jnp.uint32(5))
    return x0 ^ x1


_TINY = np.float32(np.finfo(np.float32).tiny)


def _sample_kernel(lens_ref, keys_ref, em_ref, tok_ref, plp_ref, logp_ref):
    b = pl.program_id(0)
    p = pl.program_id(1)
    nchunks = T // TCHUNK

    @pl.when(p == 0)
    def _compute_logp():
        def body(i, _):
            x = em_ref[0, :, pl.ds(i * TCHUNK, TCHUNK)]
            m = jnp.max(x, axis=0, keepdims=True)
            sh = x - m
            logp_ref[:, pl.ds(i * TCHUNK, TCHUNK)] = sh - jnp.log(
                jnp.sum(jnp.exp(sh), axis=0, keepdims=True))
            return 0
        jax.lax.fori_loop(0, nchunks, body, 0)

    k1 = keys_ref[p, 0]
    k2 = keys_ref[p, 1]
    length = lens_ref[b]
    nslices = VOCAB // 8
    vsub8_u = jax.lax.broadcasted_iota(jnp.uint32, (8, TCHUNK), 0)
    vsub8_i = jax.lax.broadcasted_iota(jnp.int32, (8, TCHUNK), 0)
    tlane_u = jax.lax.broadcasted_iota(jnp.uint32, (8, TCHUNK), 1)
    base_b = jnp.uint32(T * VOCAB) * b.astype(jnp.uint32)

    def chunk(i, acc_row):
        t0 = i * TCHUNK
        ctr0 = base_b + (tlane_u + jnp.uint32(t0)) * jnp.uint32(VOCAB) + vsub8_u
        best = bidx = blp = None
        for s in range(nslices):
            bits = _threefry_bits(k1, k2, ctr0 + jnp.uint32(8 * s))
            fb = (bits >> jnp.uint32(9)) | jnp.uint32(0x3F800000)
            f = jax.lax.bitcast_convert_type(fb, jnp.float32) - jnp.float32(1.0)
            u = jnp.maximum(f, _TINY)
            g = -jnp.log(-jnp.log(u))
            lp = logp_ref[pl.ds(8 * s, 8), pl.ds(t0, TCHUNK)]
            score = lp * jnp.float32(NBEST_SCALE) + g
            if s == 0:
                best, bidx, blp = score, vsub8_i, lp
            else:
                upd = score > best
                best = jnp.where(upd, score, best)
                bidx = jnp.where(upd, vsub8_i + (8 * s), bidx)
                blp = jnp.where(upd, lp, blp)
        for kshift in (4, 2, 1):
            rbest = pltpu.roll(best, 8 - kshift, 0)
            rbidx = pltpu.roll(bidx, 8 - kshift, 0)
            rblp = pltpu.roll(blp, 8 - kshift, 0)
            upd = (rbest > best) | ((rbest == best) & (rbidx < bidx))
            best = jnp.where(upd, rbest, best)
            bidx = jnp.where(upd, rbidx, bidx)
            blp = jnp.where(upd, rblp, blp)
        tok = bidx[0:1, :]
        lptok = blp[0:1, :]
        trow = jax.lax.broadcasted_iota(jnp.int32, (1, TCHUNK), 1) + t0
        valid = trow < length
        tok = jnp.where(valid, tok, 0)
        acc_row = acc_row + jnp.where(valid, lptok, 0.0)
        tok_ref[0, 0, pl.ds(0, 1), pl.ds(t0, TCHUNK)] = tok
        return acc_row

    acc_row = jax.lax.fori_loop(0, nchunks, chunk,
                                jnp.zeros((1, TCHUNK), jnp.float32))
    acc = jnp.sum(acc_row)
    plp_ref[0, 0, pl.ds(0, 1), :] = jnp.broadcast_to(
        acc.reshape(1, 1), (1, PPAD))


PACK = 4
W = PACK * PPAD


def _dp_kernel(rlens_ref, tok_ref, plp_ref, refbc_ref, loss_ref, d_ref):
    g = pl.program_id(0)
    refbc = refbc_ref[0]
    jr = jax.lax.broadcasted_iota(jnp.int32, (JPAD, W), 0)
    laneg = jax.lax.broadcasted_iota(jnp.int32, (1, W), 1)
    lanemask = (laneg & (PPAD - 1)) < P
    inf_row = jnp.full((1, W), BIG, jnp.int32)

    d0 = jr
    cprev0 = jnp.zeros((1, W), jnp.int32)

    def step(t, carry):
        d, cprev = carry
        c = tok_ref[0, pl.ds(t, 1), :]
        c = jnp.where(lanemask, c, 0)
        keep = (c != 0) & (c != cprev)
        cb = jnp.broadcast_to(c, (JPAD, W))
        cost = jnp.where(cb != refbc, 1, 0)
        a = d + cost
        s = jnp.concatenate([inf_row, a[:-1]], axis=0)
        z = jnp.minimum(d + 1, s) - jr
        for k in (1, 2, 4, 8, 16, 32, 64, 128, 256):
            zs = jnp.concatenate(
                [jnp.full((k, W), BIG, jnp.int32), z[:-k]], axis=0)
            z = jnp.minimum(z, zs)
        dnew = jr + z
        d = jnp.where(jnp.broadcast_to(keep, (JPAD, W)), dnew, d)
        return d, c

    d, _ = jax.lax.fori_loop(0, T, step, (d0, cprev0))
    d_ref[...] = d
    for grp in range(PACK):
        rl = rlens_ref[g * PACK + grp]
        row = d_ref[pl.ds(rl, 1), :]
        wer = row[:, grp * PPAD:(grp + 1) * PPAD].astype(jnp.float32)
        lp = plp_ref[0, pl.ds(0, 1), pl.ds(grp * PPAD, PPAD)]
        m = jnp.max(lp, axis=-1, keepdims=True)
        e = jnp.exp(lp - m)
        num = jnp.sum(e * wer)
        den = jnp.sum(e)
        loss_ref[0, pl.ds(0, 1), pl.ds(grp * PPAD, 1)] = (
            num / den).reshape(1, 1)


@jax.jit
def kernel(emissions, emissions_lengths, labels, labels_length):
    subkeys = jnp.asarray(_SUBKEYS)
    em_t = jnp.transpose(emissions, (0, 2, 1))

    tokens_bp, plp_bp = pl.pallas_call(
        _sample_kernel,
        grid=(B, P),
        in_specs=[
            pl.BlockSpec(memory_space=pltpu.SMEM),
            pl.BlockSpec(memory_space=pltpu.SMEM),
            pl.BlockSpec((1, VOCAB, T), lambda b, p: (b, 0, 0)),
        ],
        out_specs=[
            pl.BlockSpec((1, 1, 1, T), lambda b, p: (b, p, 0, 0)),
            pl.BlockSpec((1, 1, 1, PPAD), lambda b, p: (b, p, 0, 0)),
        ],
        out_shape=[
            jax.ShapeDtypeStruct((B, P, 1, T), jnp.int32),
            jax.ShapeDtypeStruct((B, P, 1, PPAD), jnp.float32),
        ],
        scratch_shapes=[pltpu.VMEM((VOCAB, T), jnp.float32)],
        compiler_params=pltpu.CompilerParams(
            dimension_semantics=("parallel", "arbitrary")),
    )(emissions_lengths, subkeys, em_t)

    tokens = jnp.pad(jnp.transpose(tokens_bp[:, :, 0, :], (0, 2, 1)),
                     ((0, 0), (0, 0), (0, PPAD - P)))
    path_logp = jnp.pad(plp_bp[:, :, 0, 0], ((0, 0), (0, PPAD - P)),
                        constant_values=-jnp.inf)[:, None, :]
    tokens = jnp.transpose(tokens.reshape(B // PACK, PACK, T, PPAD),
                           (0, 2, 1, 3)).reshape(B // PACK, T, W)
    path_logp = path_logp.reshape(B // PACK, 1, W)

    refcol = jnp.full((B, JPAD, 1), -1, jnp.int32)
    refcol = refcol.at[:, 1:L + 1, 0].set(labels)
    refbc = jnp.broadcast_to(refcol, (B, JPAD, PPAD))
    refbc = jnp.transpose(refbc.reshape(B // PACK, PACK, JPAD, PPAD),
                          (0, 2, 1, 3)).reshape(B // PACK, JPAD, W)

    loss_parts = pl.pallas_call(
        _dp_kernel,
        grid=(B // PACK,),
        in_specs=[
            pl.BlockSpec(memory_space=pltpu.SMEM),
            pl.BlockSpec((1, T, W), lambda g: (g, 0, 0)),
            pl.BlockSpec((1, 1, W), lambda g: (g, 0, 0)),
            pl.BlockSpec((1, JPAD, W), lambda g: (g, 0, 0)),
        ],
        out_specs=pl.BlockSpec((1, 1, W), lambda g: (g, 0, 0)),
        out_shape=jax.ShapeDtypeStruct((B // PACK, 1, W), jnp.float32),
        scratch_shapes=[pltpu.VMEM((JPAD, W), jnp.int32)],
        compiler_params=pltpu.CompilerParams(
            dimension_semantics=("parallel",)),
    )(labels_length, tokens, path_logp, refbc)

    return jnp.sum(loss_parts.reshape(B // PACK, PACK, PPAD)[:, :, 0])


# speedup vs baseline: 1.0427x; 1.0427x over previous
import functools

import numpy as np
import jax
import jax.numpy as jnp
from jax.experimental import pallas as pl
from jax.experimental.pallas import tpu as pltpu

B = 8
T = 2048
L = 256
VOCAB = 512
P = 100
PPAD = 128
JPAD = 264
NBEST_SCALE = 0.5
TCHUNK = 256
BIG = 1 << 20


def _np_threefry2x32(k1, k2, x0, x1):
    ks0 = np.uint32(k1)
    ks1 = np.uint32(k2)
    ks2 = np.uint32(ks0 ^ ks1 ^ np.uint32(0x1BD11BDA))

    def rotl(x, d):
        return (x << np.uint32(d)) | (x >> np.uint32(32 - d))

    def rounds(x0, x1, rots):
        for r in rots:
            x0 = (x0 + x1).astype(np.uint32)
            x1 = x0 ^ rotl(x1, r)
        return x0, x1

    x0 = (x0 + ks0).astype(np.uint32)
    x1 = (x1 + ks1).astype(np.uint32)
    x0, x1 = rounds(x0, x1, (13, 15, 26, 6))
    x0 = (x0 + ks1).astype(np.uint32)
    x1 = (x1 + ks2 + np.uint32(1)).astype(np.uint32)
    x0, x1 = rounds(x0, x1, (17, 29, 16, 24))
    x0 = (x0 + ks2).astype(np.uint32)
    x1 = (x1 + ks0 + np.uint32(2)).astype(np.uint32)
    x0, x1 = rounds(x0, x1, (13, 15, 26, 6))
    x0 = (x0 + ks0).astype(np.uint32)
    x1 = (x1 + ks1 + np.uint32(3)).astype(np.uint32)
    x0, x1 = rounds(x0, x1, (17, 29, 16, 24))
    x0 = (x0 + ks1).astype(np.uint32)
    x1 = (x1 + ks2 + np.uint32(4)).astype(np.uint32)
    x0, x1 = rounds(x0, x1, (13, 15, 26, 6))
    x0 = (x0 + ks2).astype(np.uint32)
    x1 = (x1 + ks0 + np.uint32(5)).astype(np.uint32)
    return x0, x1


def _path_subkeys():
    ks = []
    k = (np.uint32(0), np.uint32(42))
    for _ in range(P):
        b1, b2 = _np_threefry2x32(k[0], k[1],
                                  np.zeros(2, np.uint32),
                                  np.arange(2, dtype=np.uint32))
        k = (b1[0], b2[0])
        ks.append((b1[1], b2[1]))
    return np.asarray(ks, dtype=np.uint32)


_SUBKEYS = _path_subkeys()


def _threefry_bits(k1, k2, x1):
    ks0 = k1
    ks1 = k2
    ks2 = k1 ^ k2 ^ jnp.uint32(0x1BD11BDA)

    def rotl(x, d):
        return (x << jnp.uint32(d)) | (x >> jnp.uint32(32 - d))

    def rounds(x0, x1, rots):
        for r in rots:
            x0 = x0 + x1
            x1 = x0 ^ rotl(x1, r)
        return x0, x1

    x0 = jnp.broadcast_to(ks0, x1.shape)
    x1 = x1 + ks1
    x0, x1 = rounds(x0, x1, (13, 15, 26, 6))
    x0 = x0 + ks1
    x1 = x1 + (ks2 + jnp.uint32(1))
    x0, x1 = rounds(x0, x1, (17, 29, 16, 24))
    x0 = x0 + ks2
    x1 = x1 + (ks0 + jnp.uint32(2))
    x0, x1 = rounds(x0, x1, (13, 15, 26, 6))
    x0 = x0 + ks0
    x1 = x1 + (ks1 + jnp.uint32(3))
    x0, x1 = rounds(x0, x1, (17, 29, 16, 24))
    x0 = x0 + ks1
    x1 = x1 + (ks2 + jnp.uint32(4))
    x0, x1 = rounds(x0, x1, (13, 15, 26, 6))
    x0 = x0 + ks2
    x1 = x1 + (ks0 + jnp.uint32(5))
    return x0 ^ x1


_TINY = np.float32(np.finfo(np.float32).tiny)


def _sample_kernel(lens_ref, keys_ref, em_ref, tok_ref, plp_ref, logp_ref):
    b = pl.program_id(0)
    p = pl.program_id(1)
    nchunks = T // TCHUNK

    @pl.when(p == 0)
    def _compute_logp():
        def body(i, _):
            x = em_ref[0, :, pl.ds(i * TCHUNK, TCHUNK)]
            m = jnp.max(x, axis=0, keepdims=True)
            sh = x - m
            logp_ref[:, pl.ds(i * TCHUNK, TCHUNK)] = sh - jnp.log(
                jnp.sum(jnp.exp(sh), axis=0, keepdims=True))
            return 0
        jax.lax.fori_loop(0, nchunks, body, 0)

    k1 = keys_ref[p, 0]
    k2 = keys_ref[p, 1]
    length = lens_ref[b]
    nslices = VOCAB // 8
    vsub8_u = jax.lax.broadcasted_iota(jnp.uint32, (8, TCHUNK), 0)
    vsub8_i = jax.lax.broadcasted_iota(jnp.int32, (8, TCHUNK), 0)
    tlane_u = jax.lax.broadcasted_iota(jnp.uint32, (8, TCHUNK), 1)
    base_b = jnp.uint32(T * VOCAB) * b.astype(jnp.uint32)

    def chunk(i, acc_row):
        t0 = i * TCHUNK
        ctr0 = base_b + (tlane_u + jnp.uint32(t0)) * jnp.uint32(VOCAB) + vsub8_u
        best = bidx = blp = None
        for s in range(nslices):
            bits = _threefry_bits(k1, k2, ctr0 + jnp.uint32(8 * s))
            fb = (bits >> jnp.uint32(9)) | jnp.uint32(0x3F800000)
            f = jax.lax.bitcast_convert_type(fb, jnp.float32) - jnp.float32(1.0)
            u = jnp.maximum(f, _TINY)
            g = -jnp.log(-jnp.log(u))
            lp = logp_ref[pl.ds(8 * s, 8), pl.ds(t0, TCHUNK)]
            score = lp * jnp.float32(NBEST_SCALE) + g
            if s == 0:
                best, bidx, blp = score, vsub8_i, lp
            else:
                upd = score > best
                best = jnp.where(upd, score, best)
                bidx = jnp.where(upd, vsub8_i + (8 * s), bidx)
                blp = jnp.where(upd, lp, blp)
        for kshift in (4, 2, 1):
            rbest = pltpu.roll(best, 8 - kshift, 0)
            rbidx = pltpu.roll(bidx, 8 - kshift, 0)
            rblp = pltpu.roll(blp, 8 - kshift, 0)
            upd = (rbest > best) | ((rbest == best) & (rbidx < bidx))
            best = jnp.where(upd, rbest, best)
            bidx = jnp.where(upd, rbidx, bidx)
            blp = jnp.where(upd, rblp, blp)
        tok = bidx[0:1, :]
        lptok = blp[0:1, :]
        trow = jax.lax.broadcasted_iota(jnp.int32, (1, TCHUNK), 1) + t0
        valid = trow < length
        tok = jnp.where(valid, tok, 0)
        acc_row = acc_row + jnp.where(valid, lptok, 0.0)
        tok_ref[0, 0, pl.ds(0, 1), pl.ds(t0, TCHUNK)] = tok
        return acc_row

    acc_row = jax.lax.fori_loop(0, nchunks, chunk,
                                jnp.zeros((1, TCHUNK), jnp.float32))
    acc = jnp.sum(acc_row)
    plp_ref[0, 0, pl.ds(0, 1), :] = jnp.broadcast_to(
        acc.reshape(1, 1), (1, PPAD))


PACK = 1
W = PACK * PPAD


def _dp_kernel(rlens_ref, tok_ref, plp_ref, refbc_ref, loss_ref, d_ref):
    g = pl.program_id(0)
    refbc = refbc_ref[0]
    jr = jax.lax.broadcasted_iota(jnp.int32, (JPAD, W), 0)
    laneg = jax.lax.broadcasted_iota(jnp.int32, (1, W), 1)
    lanemask = (laneg & (PPAD - 1)) < P
    inf_row = jnp.full((1, W), BIG, jnp.int32)

    d0 = jnp.zeros((JPAD, W), jnp.int32)
    cprev0 = jnp.zeros((1, W), jnp.int32)

    def step(t, carry):
        d, cprev = carry
        c = tok_ref[0, pl.ds(t, 1), :]
        c = jnp.where(lanemask, c, 0)
        keep = (c != 0) & (c != cprev)
        cb = jnp.broadcast_to(c, (JPAD, W))
        a = d - (cb == refbc).astype(jnp.int32)
        s = jnp.concatenate([inf_row, a[:-1]], axis=0)
        z = jnp.minimum(d + 1, s)
        for k in (1, 2, 4, 8, 16, 32, 64, 128, 256):
            zs = jnp.concatenate(
                [jnp.full((k, W), BIG, jnp.int32), z[:-k]], axis=0)
            z = jnp.minimum(z, zs)
        d = jnp.where(jnp.broadcast_to(keep, (JPAD, W)), z, d)
        return d, c

    d, _ = jax.lax.fori_loop(0, T, step, (d0, cprev0))
    d_ref[...] = d + jr
    for grp in range(PACK):
        rl = rlens_ref[g * PACK + grp]
        row = d_ref[pl.ds(rl, 1), :]
        wer = row[:, grp * PPAD:(grp + 1) * PPAD].astype(jnp.float32)
        lp = plp_ref[0, pl.ds(0, 1), pl.ds(grp * PPAD, PPAD)]
        m = jnp.max(lp, axis=-1, keepdims=True)
        e = jnp.exp(lp - m)
        num = jnp.sum(e * wer)
        den = jnp.sum(e)
        loss_ref[0, pl.ds(0, 1), pl.ds(grp * PPAD, 1)] = (
            num / den).reshape(1, 1)


@jax.jit
def kernel(emissions, emissions_lengths, labels, labels_length):
    subkeys = jnp.asarray(_SUBKEYS)
    em_t = jnp.transpose(emissions, (0, 2, 1))

    tokens_bp, plp_bp = pl.pallas_call(
        _sample_kernel,
        grid=(B, P),
        in_specs=[
            pl.BlockSpec(memory_space=pltpu.SMEM),
            pl.BlockSpec(memory_space=pltpu.SMEM),
            pl.BlockSpec((1, VOCAB, T), lambda b, p: (b, 0, 0)),
        ],
        out_specs=[
            pl.BlockSpec((1, 1, 1, T), lambda b, p: (b, p, 0, 0)),
            pl.BlockSpec((1, 1, 1, PPAD), lambda b, p: (b, p, 0, 0)),
        ],
        out_shape=[
            jax.ShapeDtypeStruct((B, P, 1, T), jnp.int32),
            jax.ShapeDtypeStruct((B, P, 1, PPAD), jnp.float32),
        ],
        scratch_shapes=[pltpu.VMEM((VOCAB, T), jnp.float32)],
        compiler_params=pltpu.CompilerParams(
            dimension_semantics=("parallel", "arbitrary")),
    )(emissions_lengths, subkeys, em_t)

    tokens = jnp.pad(jnp.transpose(tokens_bp[:, :, 0, :], (0, 2, 1)),
                     ((0, 0), (0, 0), (0, PPAD - P)))
    path_logp = jnp.pad(plp_bp[:, :, 0, 0], ((0, 0), (0, PPAD - P)),
                        constant_values=-jnp.inf)[:, None, :]
    tokens = jnp.transpose(tokens.reshape(B // PACK, PACK, T, PPAD),
                           (0, 2, 1, 3)).reshape(B // PACK, T, W)
    path_logp = path_logp.reshape(B // PACK, 1, W)

    refcol = jnp.full((B, JPAD, 1), -1, jnp.int32)
    refcol = refcol.at[:, 0:L, 0].set(labels)
    refbc = jnp.broadcast_to(refcol, (B, JPAD, PPAD))
    refbc = jnp.transpose(refbc.reshape(B // PACK, PACK, JPAD, PPAD),
                          (0, 2, 1, 3)).reshape(B // PACK, JPAD, W)

    loss_parts = pl.pallas_call(
        _dp_kernel,
        grid=(B // PACK,),
        in_specs=[
            pl.BlockSpec(memory_space=pltpu.SMEM),
            pl.BlockSpec((1, T, W), lambda g: (g, 0, 0)),
            pl.BlockSpec((1, 1, W), lambda g: (g, 0, 0)),
            pl.BlockSpec((1, JPAD, W), lambda g: (g, 0, 0)),
        ],
        out_specs=pl.BlockSpec((1, 1, W), lambda g: (g, 0, 0)),
        out_shape=jax.ShapeDtypeStruct((B // PACK, 1, W), jnp.float32),
        scratch_shapes=[pltpu.VMEM((JPAD, W), jnp.int32)],
        compiler_params=pltpu.CompilerParams(
            dimension_semantics=("parallel",)),
    )(labels_length, tokens, path_logp, refbc)

    return jnp.sum(loss_parts.reshape(B // PACK, PACK, PPAD)[:, :, 0])
